# baseline (speedup 1.0000x reference)
import functools

import jax
import jax.numpy as jnp
from jax import lax
from jax.experimental import pallas as pl
from jax.experimental.pallas import tpu as pltpu
from jax.experimental.pallas import tpu_sc as plsc

NAT = 10000
E = 320000
D = 128
NUM_RBF = 64

N_CORES = 2
N_SUB = 16
C = 128
TOTAL_CHUNKS = 2500
E_PAD = TOTAL_CHUNKS * C
CH_MAX = 184
STAGE_ROWS = 2504
NAT_PAD = 10752
N_PHASE = 3
QUARTER = NAT_PAD // (N_CORES * N_PHASE)
ACC_ROWS = QUARTER + 128
ZERO_PER_TILE = ACC_ROWS // N_SUB
FLUSH_PER_TILE = QUARTER // N_SUB
TAIL_ROWS = NAT - NAT // FLUSH_PER_TILE * FLUSH_PER_TILE
LANES = 16
VPR = D // LANES
SEG_VPC = C // LANES


BF = 2560

_LN2 = 0.6931471805599453


def _ssp(x):
    return jnp.maximum(x, 0.0) + jnp.log1p(jnp.exp(-jnp.abs(x))) - _LN2


def _filter_body(d_ref, c_ref, g_ref, w1_ref, b1_ref, w2_ref, b2_ref, f_ref):
    d = d_ref[0]
    cen = c_ref[:]
    gam = g_ref[:]
    diff = d - cen
    ex = jnp.exp(-gam * diff * diff)
    h = lax.dot_general(ex, w1_ref[:], (((0,), (0,)), ((), ())),
                        preferred_element_type=jnp.float32)
    h = _ssp(h + b1_ref[:])
    h = jnp.dot(h, w2_ref[:], preferred_element_type=jnp.float32)
    f_ref[:] = _ssp(h + b2_ref[:])


def _filter(distances, centers, gamma, W1, b1, W2, b2):
    grid = E_PAD // BF
    return pl.pallas_call(
        _filter_body,
        grid=(grid,),
        in_specs=[
            pl.BlockSpec((1, 1, BF), lambda i: (i, 0, 0)),
            pl.BlockSpec((NUM_RBF, 1), lambda i: (0, 0)),
            pl.BlockSpec((NUM_RBF, 1), lambda i: (0, 0)),
            pl.BlockSpec((NUM_RBF, D), lambda i: (0, 0)),
            pl.BlockSpec((1, D), lambda i: (0, 0)),
            pl.BlockSpec((D, D), lambda i: (0, 0)),
            pl.BlockSpec((1, D), lambda i: (0, 0)),
        ],
        out_specs=pl.BlockSpec((BF, D), lambda i: (i, 0)),
        out_shape=jax.ShapeDtypeStruct((E_PAD, D), jnp.float32),
    )(distances.reshape(grid, 1, BF), centers.reshape(NUM_RBF, 1),
      gamma.reshape(NUM_RBF, 1), W1, b1.reshape(1, D), W2, b2.reshape(1, D))


def _sc_body(af, f, idx2d, seg2d, ranges, zeros, out,
             idx_v, seg_v, f_v, rows_v, seg_adj, acc, rng, gsem, fsem, ssem):
    cid = lax.axis_index("c")
    sid = lax.axis_index("s")

    for p in range(N_PHASE):
        if p:
            plsc.subcore_barrier()
        pltpu.sync_copy(ranges.at[cid * N_PHASE * N_SUB + p * N_SUB + sid], rng)
        rngv = rng[...]
        base = pl.multiple_of(rngv[0], 8)
        c_lo = rngv[1]
        c_hi = rngv[2]
        seg_off = (N_CORES * p + cid) * QUARTER

        pltpu.sync_copy(zeros,
                        acc.at[pl.ds(sid * ZERO_PER_TILE, ZERO_PER_TILE)])
        pltpu.sync_copy(idx2d.at[pl.ds(base, CH_MAX)], idx_v)
        pltpu.sync_copy(seg2d.at[pl.ds(base, CH_MAX)], seg_v)
        plsc.subcore_barrier()

        def _wait_scatter(b):
            pltpu.make_async_copy(
                rows_v[b], acc.at[seg_adj[b].at[0]], ssem[b]).wait()

        def _stage(t, bi, bp):
            @pl.when(t < c_hi)
            def _issue():
                @pl.when(t - 2 >= c_lo)
                def _():
                    _wait_scatter(bi)
                jr = t - base
                pltpu.async_copy(af.at[idx_v.at[jr]], rows_v[bi], gsem[bi])
                ebase = pl.multiple_of(t * C, 8)
                pltpu.async_copy(f.at[pl.ds(ebase, C)], f_v[bi], fsem[bi])

            @pl.when(t - 1 >= c_lo)
            def _process():
                jr = t - 1 - base
                for k in range(SEG_VPC):
                    sl = pl.ds(k * LANES, LANES)
                    s = seg_v[jr, sl] - seg_off
                    ok = (s >= 0) & (s < QUARTER)
                    seg_adj[bp][0, sl] = jnp.where(ok, s, QUARTER)
                ebase = pl.multiple_of((t - 1) * C, 8)
                pltpu.make_async_copy(
                    af.at[idx_v.at[jr]], rows_v[bp], gsem[bp]).wait()
                pltpu.make_async_copy(
                    f.at[pl.ds(ebase, C)], f_v[bp], fsem[bp]).wait()

                @pl.loop(0, C)
                def _row(i):
                    for k in range(VPR):
                        sl = pl.ds(k * LANES, LANES)
                        rows_v[bp][i, sl] = rows_v[bp][i, sl] * f_v[bp][i, sl]

                pltpu.async_copy(rows_v[bp], acc.at[seg_adj[bp].at[0]],
                                 ssem[bp], add=True)

        @pl.loop(c_lo, c_hi + 1)
        def _step(t):
            even = (t % 2) == 0

            @pl.when(even)
            def _():
                _stage(t, 0, 1)

            @pl.when(jnp.logical_not(even))
            def _():
                _stage(t, 1, 0)

        for d in (1, 2):
            last = c_hi - d

            @pl.when(last >= c_lo)
            def _():
                even = (last % 2) == 0

                @pl.when(even)
                def _():
                    _wait_scatter(0)

                @pl.when(jnp.logical_not(even))
                def _():
                    _wait_scatter(1)

        plsc.subcore_barrier()
        row_base = pl.multiple_of(sid * FLUSH_PER_TILE, 8)
        out_base = pl.multiple_of(seg_off + sid * FLUSH_PER_TILE, 8)
        left = NAT - out_base

        @pl.when(left >= FLUSH_PER_TILE)
        def _flush_full():
            pltpu.sync_copy(acc.at[pl.ds(row_base, FLUSH_PER_TILE)],
                            out.at[pl.ds(out_base, FLUSH_PER_TILE)])

        @pl.when((left > 0) & (left < FLUSH_PER_TILE))
        def _flush_tail():
            pltpu.sync_copy(acc.at[pl.ds(row_base, TAIL_ROWS)],
                            out.at[pl.ds(out_base, TAIL_ROWS)])


_sc_scatter = functools.partial(
    pl.kernel,
    out_type=jax.ShapeDtypeStruct((NAT, D), jnp.float32),
    mesh=plsc.VectorSubcoreMesh(core_axis_name="c", subcore_axis_name="s"),
    scratch_types=[
        pltpu.VMEM((CH_MAX, C), jnp.int32),
        pltpu.VMEM((CH_MAX, C), jnp.int32),
        [pltpu.VMEM((C, D), jnp.float32)] * 2,
        [pltpu.VMEM((C, D), jnp.float32)] * 2,
        [pltpu.VMEM((8, C), jnp.int32)] * 2,
        pltpu.VMEM_SHARED((ACC_ROWS, D), jnp.float32),
        pltpu.VMEM((16,), jnp.int32),
        [pltpu.SemaphoreType.DMA] * 2,
        [pltpu.SemaphoreType.DMA] * 2,
        [pltpu.SemaphoreType.DMA] * 2,
    ],
)(_sc_body)


def _tile_ranges(seg_pad):
    qb = (jnp.arange(1, N_CORES * N_PHASE, dtype=jnp.int32) * QUARTER)
    bounds = jnp.sum(seg_pad[None, :] < qb[:, None], axis=1).astype(jnp.int32)
    zero = jnp.zeros((), jnp.int32)
    full = jnp.full((), E, jnp.int32)
    b = [zero] + [bounds[i] for i in range(N_CORES * N_PHASE - 1)] + [full]
    rows = []
    for c in range(N_CORES):
        for p in range(N_PHASE):
            q = N_CORES * p + c
            lo_q = (b[q] // C).astype(jnp.int32)
            hi_q = ((b[q + 1] + C - 1) // C).astype(jnp.int32)
            n = hi_q - lo_q
            for t in range(N_SUB):
                b_lo = lo_q + (n * t // N_SUB) // 8 * 8
                b_hi = jnp.where(t == N_SUB - 1, hi_q,
                                 lo_q + (n * (t + 1) // N_SUB) // 8 * 8)
                base = jnp.clip(b_lo // 8 * 8, 0, STAGE_ROWS - CH_MAX)
                z = jnp.zeros((), jnp.int32)
                rows.append(jnp.stack([base, b_lo, b_hi] + [z] * 13))
    return jnp.stack(rows).astype(jnp.int32)


def kernel(atom_features, distances, idx_j, seg_i, centers, gamma, W1, b1, W2, b2):
    idx32 = idx_j.astype(jnp.int32)
    seg32 = seg_i.astype(jnp.int32)
    f = _filter(distances, centers, gamma, W1, b1, W2, b2)
    ranges = _tile_ranges(seg32)
    pad = jnp.zeros((STAGE_ROWS - TOTAL_CHUNKS, C), jnp.int32)
    idx2d = jnp.concatenate([idx32.reshape(TOTAL_CHUNKS, C), pad])
    seg2d = jnp.concatenate([seg32.reshape(TOTAL_CHUNKS, C), pad])
    zeros = jnp.zeros((ZERO_PER_TILE, D), jnp.float32)
    return _sc_scatter(atom_features, f, idx2d, seg2d, ranges, zeros)


# speedup vs baseline: 1.0719x; 1.0719x over previous
import functools

import jax
import jax.numpy as jnp
from jax import lax
from jax.experimental import pallas as pl
from jax.experimental.pallas import tpu as pltpu
from jax.experimental.pallas import tpu_sc as plsc

NAT = 10000
E = 320000
D = 128
NUM_RBF = 64

N_CORES = 2
N_SUB = 16
C = 128
TOTAL_CHUNKS = 2500
E_PAD = TOTAL_CHUNKS * C
CH_MAX = 184
STAGE_ROWS = 2504
NAT_PAD = 10752
N_PHASE = 3
QUARTER = NAT_PAD // (N_CORES * N_PHASE)
ACC_ROWS = QUARTER + 128
ZERO_PER_TILE = ACC_ROWS // N_SUB
FLUSH_PER_TILE = QUARTER // N_SUB
TAIL_ROWS = NAT - NAT // FLUSH_PER_TILE * FLUSH_PER_TILE
LANES = 16
VPR = D // LANES
SEG_VPC = C // LANES


BF = 2560

_LN2 = 0.6931471805599453


def _ssp(x):
    return jnp.maximum(x, 0.0) + jnp.log1p(jnp.exp(-jnp.abs(x))) - _LN2


def _filter_body(d_ref, c_ref, g_ref, w1_ref, b1_ref, w2_ref, b2_ref, f_ref):
    d = d_ref[0]
    cen = c_ref[:]
    gam = g_ref[:]
    diff = d - cen
    ex = jnp.exp(-gam * diff * diff)
    h = lax.dot_general(ex, w1_ref[:], (((0,), (0,)), ((), ())),
                        preferred_element_type=jnp.float32)
    h = _ssp(h + b1_ref[:])
    h = jnp.dot(h, w2_ref[:], preferred_element_type=jnp.float32)
    f_ref[:] = _ssp(h + b2_ref[:])


def _filter(distances, centers, gamma, W1, b1, W2, b2):
    grid = E_PAD // BF
    return pl.pallas_call(
        _filter_body,
        grid=(grid,),
        in_specs=[
            pl.BlockSpec((1, 1, BF), lambda i: (i, 0, 0)),
            pl.BlockSpec((NUM_RBF, 1), lambda i: (0, 0)),
            pl.BlockSpec((NUM_RBF, 1), lambda i: (0, 0)),
            pl.BlockSpec((NUM_RBF, D), lambda i: (0, 0)),
            pl.BlockSpec((1, D), lambda i: (0, 0)),
            pl.BlockSpec((D, D), lambda i: (0, 0)),
            pl.BlockSpec((1, D), lambda i: (0, 0)),
        ],
        out_specs=pl.BlockSpec((BF, D), lambda i: (i, 0)),
        out_shape=jax.ShapeDtypeStruct((E_PAD, D), jnp.float32),
    )(distances.reshape(grid, 1, BF), centers.reshape(NUM_RBF, 1),
      gamma.reshape(NUM_RBF, 1), W1, b1.reshape(1, D), W2, b2.reshape(1, D))


def _sc_body(af, f, idx2d, seg2d, ranges, zeros, out,
             idx_v, seg_v, f_v, rows_v, seg_adj, acc, rng, gsem, fsem, ssem):
    cid = lax.axis_index("c")
    sid = lax.axis_index("s")

    for p in range(N_PHASE):
        if p:
            plsc.subcore_barrier()
        pltpu.sync_copy(ranges.at[cid * N_PHASE * N_SUB + p * N_SUB + sid], rng)
        rngv = rng[...]
        base = pl.multiple_of(rngv[0], 8)
        c_lo = rngv[1]
        c_hi = rngv[2]
        seg_off = (N_CORES * p + cid) * QUARTER

        pltpu.sync_copy(zeros,
                        acc.at[pl.ds(sid * ZERO_PER_TILE, ZERO_PER_TILE)])
        pltpu.sync_copy(idx2d.at[pl.ds(base, CH_MAX)], idx_v)
        pltpu.sync_copy(seg2d.at[pl.ds(base, CH_MAX)], seg_v)
        plsc.subcore_barrier()

        def _wait_scatter(b):
            pltpu.make_async_copy(
                rows_v[b], acc.at[seg_adj[b].at[0]], ssem[b]).wait()

        def _stage(t, bi, bp):
            @pl.when(t < c_hi)
            def _issue():
                @pl.when(t - 2 >= c_lo)
                def _():
                    _wait_scatter(bi)
                jr = t - base
                pltpu.async_copy(af.at[idx_v.at[jr]], rows_v[bi], gsem[bi])
                ebase = pl.multiple_of(t * C, 8)
                pltpu.async_copy(f.at[pl.ds(ebase, C)], f_v[bi], fsem[bi])

            @pl.when(t - 1 >= c_lo)
            def _process():
                jr = t - 1 - base
                for k in range(SEG_VPC):
                    sl = pl.ds(k * LANES, LANES)
                    s = seg_v[jr, sl] - seg_off
                    ok = (s >= 0) & (s < QUARTER)
                    seg_adj[bp][0, sl] = jnp.where(ok, s, QUARTER)
                ebase = pl.multiple_of((t - 1) * C, 8)
                pltpu.make_async_copy(
                    af.at[idx_v.at[jr]], rows_v[bp], gsem[bp]).wait()
                pltpu.make_async_copy(
                    f.at[pl.ds(ebase, C)], f_v[bp], fsem[bp]).wait()

                @pl.loop(0, C)
                def _row(i):
                    for k in range(VPR):
                        sl = pl.ds(k * LANES, LANES)
                        rows_v[bp][i, sl] = rows_v[bp][i, sl] * f_v[bp][i, sl]

                pltpu.async_copy(rows_v[bp], acc.at[seg_adj[bp].at[0]],
                                 ssem[bp], add=True)

        @pl.loop(c_lo, c_hi + 1)
        def _step(t):
            even = (t % 2) == 0

            @pl.when(even)
            def _():
                _stage(t, 0, 1)

            @pl.when(jnp.logical_not(even))
            def _():
                _stage(t, 1, 0)

        for d in (1, 2):
            last = c_hi - d

            @pl.when(last >= c_lo)
            def _():
                even = (last % 2) == 0

                @pl.when(even)
                def _():
                    _wait_scatter(0)

                @pl.when(jnp.logical_not(even))
                def _():
                    _wait_scatter(1)

        plsc.subcore_barrier()
        row_base = pl.multiple_of(sid * FLUSH_PER_TILE, 8)
        out_base = pl.multiple_of(seg_off + sid * FLUSH_PER_TILE, 8)
        left = NAT - out_base

        @pl.when(left >= FLUSH_PER_TILE)
        def _flush_full():
            pltpu.sync_copy(acc.at[pl.ds(row_base, FLUSH_PER_TILE)],
                            out.at[pl.ds(out_base, FLUSH_PER_TILE)])

        @pl.when((left > 0) & (left < FLUSH_PER_TILE))
        def _flush_tail():
            pltpu.sync_copy(acc.at[pl.ds(row_base, TAIL_ROWS)],
                            out.at[pl.ds(out_base, TAIL_ROWS)])


_sc_scatter = functools.partial(
    pl.kernel,
    out_type=jax.ShapeDtypeStruct((NAT, D), jnp.float32),
    mesh=plsc.VectorSubcoreMesh(core_axis_name="c", subcore_axis_name="s"),
    scratch_types=[
        pltpu.VMEM((CH_MAX, C), jnp.int32),
        pltpu.VMEM((CH_MAX, C), jnp.int32),
        [pltpu.VMEM((C, D), jnp.float32)] * 2,
        [pltpu.VMEM((C, D), jnp.float32)] * 2,
        [pltpu.VMEM((8, C), jnp.int32)] * 2,
        pltpu.VMEM_SHARED((ACC_ROWS, D), jnp.float32),
        pltpu.VMEM((16,), jnp.int32),
        [pltpu.SemaphoreType.DMA] * 2,
        [pltpu.SemaphoreType.DMA] * 2,
        [pltpu.SemaphoreType.DMA] * 2,
    ],
)(_sc_body)


def _tile_ranges(seg):
    qb = (jnp.arange(1, N_CORES * N_PHASE, dtype=jnp.int32) * QUARTER)
    bounds = jnp.sum(seg[None, :] < qb[:, None], axis=1).astype(jnp.int32)
    b = jnp.concatenate([jnp.zeros((1,), jnp.int32), bounds,
                         jnp.full((1,), E, jnp.int32)])
    q_idx = (jnp.arange(N_PHASE, dtype=jnp.int32)[None, :] * N_CORES
             + jnp.arange(N_CORES, dtype=jnp.int32)[:, None])
    lo = b[q_idx] // C
    hi = (b[q_idx + 1] + C - 1) // C
    n = (hi - lo)[..., None]
    t = jnp.arange(N_SUB, dtype=jnp.int32)
    b_lo = lo[..., None] + (n * t // N_SUB) // 8 * 8
    b_hi = jnp.where(t == N_SUB - 1, hi[..., None],
                     lo[..., None] + (n * (t + 1) // N_SUB) // 8 * 8)
    base = jnp.clip(b_lo // 8 * 8, 0, (STAGE_ROWS - CH_MAX) // 8 * 8)
    z = jnp.zeros_like(base)
    rows = jnp.stack([base, b_lo, b_hi] + [z] * 13, axis=-1)
    return rows.reshape(N_CORES * N_PHASE * N_SUB, 16).astype(jnp.int32)


def kernel(atom_features, distances, idx_j, seg_i, centers, gamma, W1, b1, W2, b2):
    idx32 = idx_j.astype(jnp.int32)
    seg32 = seg_i.astype(jnp.int32)
    f = _filter(distances, centers, gamma, W1, b1, W2, b2)
    ranges = _tile_ranges(seg32)
    pad = jnp.zeros((STAGE_ROWS - TOTAL_CHUNKS, C), jnp.int32)
    idx2d = jnp.concatenate([idx32.reshape(TOTAL_CHUNKS, C), pad])
    seg2d = jnp.concatenate([seg32.reshape(TOTAL_CHUNKS, C), pad])
    zeros = jnp.zeros((ZERO_PER_TILE, D), jnp.float32)
    return _sc_scatter(atom_features, f, idx2d, seg2d, ranges, zeros)


# speedup vs baseline: 1.1828x; 1.1035x over previous
import functools

import jax
import jax.numpy as jnp
from jax import lax
from jax.experimental import pallas as pl
from jax.experimental.pallas import tpu as pltpu
from jax.experimental.pallas import tpu_sc as plsc

NAT = 10000
E = 320000
D = 128
NUM_RBF = 64

N_CORES = 2
N_SUB = 16
C = 128
TOTAL_CHUNKS = 2500
E_PAD = TOTAL_CHUNKS * C
CH_MAX = 184
STAGE_ROWS = 2504
NAT_PAD = 10752
N_PHASE = 3
QUARTER = NAT_PAD // (N_CORES * N_PHASE)
ACC_ROWS = QUARTER + 128
ZERO_PER_TILE = ACC_ROWS // N_SUB
FLUSH_PER_TILE = QUARTER // N_SUB
TAIL_ROWS = NAT - NAT // FLUSH_PER_TILE * FLUSH_PER_TILE
LANES = 16
VPR = D // LANES
SEG_VPC = C // LANES


BF = 2560

_LN2 = 0.6931471805599453


def _ssp(x):
    xb = x.astype(jnp.bfloat16)
    corr = jnp.log1p(jnp.exp(-jnp.abs(xb))).astype(jnp.float32)
    return jnp.maximum(x, 0.0) + corr - _LN2


def _filter_body(d_ref, c_ref, g_ref, w1_ref, b1_ref, w2_ref, b2_ref, f_ref):
    d = d_ref[0]
    cen = c_ref[:]
    gam = g_ref[:]
    diff = d - cen
    ex = jnp.exp(-gam * diff * diff)
    h = lax.dot_general(ex, w1_ref[:], (((0,), (0,)), ((), ())),
                        preferred_element_type=jnp.float32)
    h = _ssp(h + b1_ref[:])
    h = jnp.dot(h, w2_ref[:], preferred_element_type=jnp.float32)
    f_ref[:] = _ssp(h + b2_ref[:])


def _filter(distances, centers, gamma, W1, b1, W2, b2):
    grid = E_PAD // BF
    return pl.pallas_call(
        _filter_body,
        grid=(grid,),
        in_specs=[
            pl.BlockSpec((1, 1, BF), lambda i: (i, 0, 0)),
            pl.BlockSpec((NUM_RBF, 1), lambda i: (0, 0)),
            pl.BlockSpec((NUM_RBF, 1), lambda i: (0, 0)),
            pl.BlockSpec((NUM_RBF, D), lambda i: (0, 0)),
            pl.BlockSpec((1, D), lambda i: (0, 0)),
            pl.BlockSpec((D, D), lambda i: (0, 0)),
            pl.BlockSpec((1, D), lambda i: (0, 0)),
        ],
        out_specs=pl.BlockSpec((BF, D), lambda i: (i, 0)),
        out_shape=jax.ShapeDtypeStruct((E_PAD, D), jnp.float32),
    )(distances.reshape(grid, 1, BF), centers.reshape(NUM_RBF, 1),
      gamma.reshape(NUM_RBF, 1), W1, b1.reshape(1, D), W2, b2.reshape(1, D))


def _sc_body(af, f, idx2d, seg2d, ranges, zeros, out,
             idx_v, seg_v, f_v, rows_v, seg_adj, acc, rng, gsem, fsem, ssem):
    cid = lax.axis_index("c")
    sid = lax.axis_index("s")

    for p in range(N_PHASE):
        if p:
            plsc.subcore_barrier()
        pltpu.sync_copy(ranges.at[cid * N_PHASE * N_SUB + p * N_SUB + sid], rng)
        rngv = rng[...]
        base = pl.multiple_of(rngv[0], 8)
        c_lo = rngv[1]
        c_hi = rngv[2]
        seg_off = (N_CORES * p + cid) * QUARTER

        pltpu.sync_copy(zeros,
                        acc.at[pl.ds(sid * ZERO_PER_TILE, ZERO_PER_TILE)])
        pltpu.sync_copy(idx2d.at[pl.ds(base, CH_MAX)], idx_v)
        pltpu.sync_copy(seg2d.at[pl.ds(base, CH_MAX)], seg_v)
        plsc.subcore_barrier()

        def _wait_scatter(b):
            pltpu.make_async_copy(
                rows_v[b], acc.at[seg_adj[b].at[0]], ssem[b]).wait()

        def _stage(t, bi, bp):
            @pl.when(t < c_hi)
            def _issue():
                @pl.when(t - 2 >= c_lo)
                def _():
                    _wait_scatter(bi)
                jr = t - base
                pltpu.async_copy(af.at[idx_v.at[jr]], rows_v[bi], gsem[bi])
                ebase = pl.multiple_of(t * C, 8)
                pltpu.async_copy(f.at[pl.ds(ebase, C)], f_v[bi], fsem[bi])

            @pl.when(t - 1 >= c_lo)
            def _process():
                jr = t - 1 - base
                for k in range(SEG_VPC):
                    sl = pl.ds(k * LANES, LANES)
                    s = seg_v[jr, sl] - seg_off
                    ok = (s >= 0) & (s < QUARTER)
                    seg_adj[bp][0, sl] = jnp.where(ok, s, QUARTER)
                ebase = pl.multiple_of((t - 1) * C, 8)
                pltpu.make_async_copy(
                    af.at[idx_v.at[jr]], rows_v[bp], gsem[bp]).wait()
                pltpu.make_async_copy(
                    f.at[pl.ds(ebase, C)], f_v[bp], fsem[bp]).wait()

                @pl.loop(0, C)
                def _row(i):
                    for k in range(VPR):
                        sl = pl.ds(k * LANES, LANES)
                        rows_v[bp][i, sl] = rows_v[bp][i, sl] * f_v[bp][i, sl]

                pltpu.async_copy(rows_v[bp], acc.at[seg_adj[bp].at[0]],
                                 ssem[bp], add=True)

        @pl.loop(c_lo, c_hi + 1)
        def _step(t):
            even = (t % 2) == 0

            @pl.when(even)
            def _():
                _stage(t, 0, 1)

            @pl.when(jnp.logical_not(even))
            def _():
                _stage(t, 1, 0)

        for d in (1, 2):
            last = c_hi - d

            @pl.when(last >= c_lo)
            def _():
                even = (last % 2) == 0

                @pl.when(even)
                def _():
                    _wait_scatter(0)

                @pl.when(jnp.logical_not(even))
                def _():
                    _wait_scatter(1)

        plsc.subcore_barrier()
        row_base = pl.multiple_of(sid * FLUSH_PER_TILE, 8)
        out_base = pl.multiple_of(seg_off + sid * FLUSH_PER_TILE, 8)
        left = NAT - out_base

        @pl.when(left >= FLUSH_PER_TILE)
        def _flush_full():
            pltpu.sync_copy(acc.at[pl.ds(row_base, FLUSH_PER_TILE)],
                            out.at[pl.ds(out_base, FLUSH_PER_TILE)])

        @pl.when((left > 0) & (left < FLUSH_PER_TILE))
        def _flush_tail():
            pltpu.sync_copy(acc.at[pl.ds(row_base, TAIL_ROWS)],
                            out.at[pl.ds(out_base, TAIL_ROWS)])


_sc_scatter = functools.partial(
    pl.kernel,
    out_type=jax.ShapeDtypeStruct((NAT, D), jnp.float32),
    mesh=plsc.VectorSubcoreMesh(core_axis_name="c", subcore_axis_name="s"),
    scratch_types=[
        pltpu.VMEM((CH_MAX, C), jnp.int32),
        pltpu.VMEM((CH_MAX, C), jnp.int32),
        [pltpu.VMEM((C, D), jnp.float32)] * 2,
        [pltpu.VMEM((C, D), jnp.float32)] * 2,
        [pltpu.VMEM((8, C), jnp.int32)] * 2,
        pltpu.VMEM_SHARED((ACC_ROWS, D), jnp.float32),
        pltpu.VMEM((16,), jnp.int32),
        [pltpu.SemaphoreType.DMA] * 2,
        [pltpu.SemaphoreType.DMA] * 2,
        [pltpu.SemaphoreType.DMA] * 2,
    ],
)(_sc_body)


def _tile_ranges(seg):
    qb = (jnp.arange(1, N_CORES * N_PHASE, dtype=jnp.int32) * QUARTER)
    bounds = jnp.sum(seg[None, :] < qb[:, None], axis=1).astype(jnp.int32)
    b = jnp.concatenate([jnp.zeros((1,), jnp.int32), bounds,
                         jnp.full((1,), E, jnp.int32)])
    q_idx = (jnp.arange(N_PHASE, dtype=jnp.int32)[None, :] * N_CORES
             + jnp.arange(N_CORES, dtype=jnp.int32)[:, None])
    lo = b[q_idx] // C
    hi = (b[q_idx + 1] + C - 1) // C
    n = (hi - lo)[..., None]
    t = jnp.arange(N_SUB, dtype=jnp.int32)
    b_lo = lo[..., None] + (n * t // N_SUB) // 8 * 8
    b_hi = jnp.where(t == N_SUB - 1, hi[..., None],
                     lo[..., None] + (n * (t + 1) // N_SUB) // 8 * 8)
    base = jnp.clip(b_lo // 8 * 8, 0, (STAGE_ROWS - CH_MAX) // 8 * 8)
    z = jnp.zeros_like(base)
    rows = jnp.stack([base, b_lo, b_hi] + [z] * 13, axis=-1)
    return rows.reshape(N_CORES * N_PHASE * N_SUB, 16).astype(jnp.int32)


def kernel(atom_features, distances, idx_j, seg_i, centers, gamma, W1, b1, W2, b2):
    idx32 = idx_j.astype(jnp.int32)
    seg32 = seg_i.astype(jnp.int32)
    f = _filter(distances, centers, gamma, W1, b1, W2, b2)
    ranges = _tile_ranges(seg32)
    pad = jnp.zeros((STAGE_ROWS - TOTAL_CHUNKS, C), jnp.int32)
    idx2d = jnp.concatenate([idx32.reshape(TOTAL_CHUNKS, C), pad])
    seg2d = jnp.concatenate([seg32.reshape(TOTAL_CHUNKS, C), pad])
    zeros = jnp.zeros((ZERO_PER_TILE, D), jnp.float32)
    return _sc_scatter(atom_features, f, idx2d, seg2d, ranges, zeros)
